# baseline (speedup 1.0000x reference)
import functools

import jax
import jax.numpy as jnp
from jax import lax
from jax.experimental import pallas as pl
from jax.experimental.pallas import tpu as pltpu
from jax.experimental.pallas import tpu_sc as plsc

_B = 4
_S = 4
_D = 2048
_NC = 1
_NS = 16
_NW = _NC * _NS
_CHUNK = (_B * _S * _D) // _NW
_PER_ROW = _D // _CHUNK

_mesh = plsc.ScalarSubcoreMesh(axis_name="c", num_cores=2)
_BH = _B // 2


@functools.partial(
    pl.kernel,
    mesh=_mesh,
    out_type=jax.ShapeDtypeStruct((_B, _S, _D), jnp.float32),
    scratch_types=[
        pltpu.VMEM_SHARED((_BH, _S, _D), jnp.float32),
        pltpu.SemaphoreType.DMA,
    ],
)
def _gather_head(x_hbm, out_hbm, stage, sem):
    del sem
    c = lax.axis_index("c")
    base = c * _BH
    pltpu.sync_copy(x_hbm.at[pl.ds(base, _BH), pl.ds(0, _S), 0], stage)
    pltpu.sync_copy(stage, out_hbm.at[pl.ds(base, _BH)])


def kernel(x):
    return _gather_head(x)
